# baseline (speedup 1.0000x reference)
import jax
import jax.numpy as jnp
from jax import lax
from jax.experimental import pallas as pl
from jax.experimental.pallas import tpu as pltpu

B = 16
L = 4096
LP1 = L + 1
START_TOKEN = 1.0
STOP_TOKEN = 2.0

GB = 2
BB = B // GB
C = 2048
GC = 3


def _collate_body(x_ref, inp_ref, inp_len_ref, tgt_ref, tgt_len_ref, carry_ref):
    j = pl.program_id(1)
    x0 = x_ref[:, 0, :]
    x1 = x_ref[:, 1, :]
    lane = lax.broadcasted_iota(jnp.int32, (BB, C), 1)

    prev = carry_ref[:, 127:128]
    boundary = jnp.where(j == 0, jnp.float32(START_TOKEN), prev)
    rot = jnp.concatenate([x0[:, C - 1:], x0[:, :C - 1]], axis=1)
    inp_ref[...] = jnp.where(lane == 0, boundary, rot)
    carry_ref[...] = x0[:, C - 128:]

    tgt_ref[...] = jnp.where(j >= GC - 1, jnp.float32(STOP_TOKEN), x1)

    inp_len_ref[...] = jnp.full((B,), LP1, jnp.int32)
    tgt_len_ref[...] = jnp.full((B,), LP1, jnp.int32)


@jax.jit
def kernel(samples):
    return pl.pallas_call(
        _collate_body,
        grid=(GB, GC),
        in_specs=[
            pl.BlockSpec((BB, 2, C), lambda i, j: (i, 0, jnp.minimum(j, 1))),
        ],
        out_specs=(
            pl.BlockSpec((BB, C), lambda i, j: (i, j)),
            pl.BlockSpec((B,), lambda i, j: (0,)),
            pl.BlockSpec((BB, C), lambda i, j: (i, j)),
            pl.BlockSpec((B,), lambda i, j: (0,)),
        ),
        out_shape=(
            jax.ShapeDtypeStruct((B, LP1), jnp.float32),
            jax.ShapeDtypeStruct((B,), jnp.int32),
            jax.ShapeDtypeStruct((B, LP1), jnp.float32),
            jax.ShapeDtypeStruct((B,), jnp.int32),
        ),
        scratch_shapes=[pltpu.VMEM((BB, 128), jnp.float32)],
    )(samples)


# speedup vs baseline: 1.7995x; 1.7995x over previous
import functools

import jax
import jax.numpy as jnp
from jax.experimental import pallas as pl
from jax.experimental.pallas import tpu as pltpu

B = 16
L = 4096
LP1 = L + 1
START_TOKEN = 1.0
STOP_TOKEN = 2.0

GRID = 2
BB = B // GRID


def _collate_body(x_ref, inp_ref, inp_len_ref, tgt_ref, tgt_len_ref):
    x0 = x_ref[:, 0, :]
    x1 = x_ref[:, 1, :]
    start = jnp.full((BB, 1), START_TOKEN, jnp.float32)
    stop = jnp.full((BB, 1), STOP_TOKEN, jnp.float32)
    inp_ref[...] = jnp.concatenate([start, x0], axis=1)
    tgt_ref[...] = jnp.concatenate([x1, stop], axis=1)
    inp_len_ref[...] = jnp.full((B,), LP1, jnp.int32)
    tgt_len_ref[...] = jnp.full((B,), LP1, jnp.int32)


@jax.jit
def kernel(samples):
    return pl.pallas_call(
        _collate_body,
        grid=(GRID,),
        in_specs=[pl.BlockSpec((BB, 2, L), lambda i: (i, 0, 0))],
        out_specs=(
            pl.BlockSpec((BB, LP1), lambda i: (i, 0)),
            pl.BlockSpec((B,), lambda i: (0,)),
            pl.BlockSpec((BB, LP1), lambda i: (i, 0)),
            pl.BlockSpec((B,), lambda i: (0,)),
        ),
        out_shape=(
            jax.ShapeDtypeStruct((B, LP1), jnp.float32),
            jax.ShapeDtypeStruct((B,), jnp.int32),
            jax.ShapeDtypeStruct((B, LP1), jnp.float32),
            jax.ShapeDtypeStruct((B,), jnp.int32),
        ),
    )(samples)
